# baseline (speedup 1.0000x reference)
import functools

import jax
import jax.numpy as jnp
from jax import lax
from jax.experimental import pallas as pl
from jax.experimental.pallas import tpu as pltpu
from jax.experimental.pallas import tpu_sc as plsc

_TB = 256
_T = 128
_LANES = 128
_NEG = -1e30


def _gate_body(x_ref, wg_ref, bg_ref, i1_ref, i2_ref, p1_ref, p2_ref,
               imp_ref, load_ref):
    g = pl.program_id(0)
    scores = jnp.dot(x_ref[...], wg_ref[...],
                     preferred_element_type=jnp.float32) + bg_ref[...]
    m = jnp.max(scores, axis=1, keepdims=True)
    ex = jnp.exp(scores - m)
    s = jnp.sum(ex, axis=1, keepdims=True)
    probs = ex / s
    iota = lax.broadcasted_iota(jnp.int32, probs.shape, 1)
    m1 = jnp.max(probs, axis=1, keepdims=True)
    a1 = jnp.min(jnp.where(probs == m1, iota, 1 << 30), axis=1, keepdims=True)
    probs_m = jnp.where(iota == a1, -1.0, probs)
    m2 = jnp.max(probs_m, axis=1, keepdims=True)
    a2 = jnp.min(jnp.where(probs_m == m2, iota, 1 << 30), axis=1,
                 keepdims=True)
    e2 = jnp.exp(m2 - m1)
    denom = 1.0 + e2
    i1_ref[...] = a1
    i2_ref[...] = a2
    p1_ref[...] = 1.0 / denom
    p2_ref[...] = e2 / denom
    imp_blk = jnp.sum(probs, axis=0, keepdims=True)
    load_blk = jnp.sum((probs > 0).astype(jnp.float32), axis=0, keepdims=True)

    @pl.when(g == 0)
    def _():
        imp_ref[...] = imp_blk
        load_ref[...] = load_blk

    @pl.when(g != 0)
    def _():
        imp_ref[...] += imp_blk
        load_ref[...] += load_blk


def _gate(x_flat, Wg, bg):
    n, d = x_flat.shape
    e = Wg.shape[0]
    wgp = jnp.zeros((d, _LANES), jnp.float32).at[:, :e].set(Wg.T)
    bgp = jnp.full((1, _LANES), _NEG, jnp.float32).at[0, :e].set(bg)
    ntb = n // _TB
    return pl.pallas_call(
        _gate_body,
        grid=(ntb,),
        in_specs=[
            pl.BlockSpec((_TB, d), lambda g: (g, 0)),
            pl.BlockSpec((d, _LANES), lambda g: (0, 0)),
            pl.BlockSpec((1, _LANES), lambda g: (0, 0)),
        ],
        out_specs=[
            pl.BlockSpec((_TB, 1), lambda g: (g, 0)),
            pl.BlockSpec((_TB, 1), lambda g: (g, 0)),
            pl.BlockSpec((_TB, 1), lambda g: (g, 0)),
            pl.BlockSpec((_TB, 1), lambda g: (g, 0)),
            pl.BlockSpec((1, _LANES), lambda g: (0, 0)),
            pl.BlockSpec((1, _LANES), lambda g: (0, 0)),
        ],
        out_shape=[
            jax.ShapeDtypeStruct((n, 1), jnp.int32),
            jax.ShapeDtypeStruct((n, 1), jnp.int32),
            jax.ShapeDtypeStruct((n, 1), jnp.float32),
            jax.ShapeDtypeStruct((n, 1), jnp.float32),
            jax.ShapeDtypeStruct((1, _LANES), jnp.float32),
            jax.ShapeDtypeStruct((1, _LANES), jnp.float32),
        ],
    )(x_flat, wgp, bgp)


_NC = 2
_NS = 16
_NW = _NC * _NS
_CH = 32


def _sc_gather(table, idx):
    np_, d = idx.shape[0], table.shape[1]
    per_w = np_ // _NW
    n_ch = per_w // _CH
    mesh = plsc.VectorSubcoreMesh(core_axis_name="core",
                                  subcore_axis_name="subcore")

    @functools.partial(
        pl.kernel,
        out_type=jax.ShapeDtypeStruct((np_, d), table.dtype),
        mesh=mesh,
        scratch_types=[
            pltpu.VMEM((per_w,), jnp.int32),
            pltpu.VMEM((_CH, d), jnp.float32),
            pltpu.VMEM((_CH, d), jnp.float32),
            pltpu.SemaphoreType.DMA,
            pltpu.SemaphoreType.DMA,
        ])
    def k(x_hbm, i_hbm, o_hbm, idx_v, row0_v, row1_v, sem0, sem1):
        wid = lax.axis_index("subcore") * _NC + lax.axis_index("core")
        base = wid * per_w
        pltpu.sync_copy(i_hbm.at[pl.ds(base, per_w)], idx_v)
        bufs = (row0_v, row1_v)
        sems = (sem0, sem1)
        cps = []
        for c in range(n_ch):
            b = c % 2
            cp = pltpu.async_copy(
                x_hbm.at[idx_v.at[pl.ds(c * _CH, _CH)]], bufs[b], sems[b])
            if c > 0:
                cps[c - 1].wait()
                pltpu.sync_copy(bufs[1 - b],
                                o_hbm.at[pl.ds(base + (c - 1) * _CH, _CH)])
            cps.append(cp)
        cps[-1].wait()
        pltpu.sync_copy(bufs[(n_ch - 1) % 2],
                        o_hbm.at[pl.ds(base + (n_ch - 1) * _CH, _CH)])

    return k(table, idx)


def _sc_combine(opad, pos0, pos1):
    n = pos0.shape[0]
    d = opad.shape[1]
    per_w = n // _NW
    n_ch = per_w // _CH
    mesh = plsc.VectorSubcoreMesh(core_axis_name="core",
                                  subcore_axis_name="subcore")

    @functools.partial(
        pl.kernel,
        out_type=jax.ShapeDtypeStruct((n, d), opad.dtype),
        mesh=mesh,
        scratch_types=[
            pltpu.VMEM((per_w,), jnp.int32),
            pltpu.VMEM((per_w,), jnp.int32),
            pltpu.VMEM((_CH, d), jnp.float32),
            pltpu.VMEM((_CH, d), jnp.float32),
            pltpu.SemaphoreType.DMA,
            pltpu.SemaphoreType.DMA,
        ])
    def k(x_hbm, i0_hbm, i1_hbm, o_hbm, i0_v, i1_v, row0_v, row1_v,
          sem0, sem1):
        wid = lax.axis_index("subcore") * _NC + lax.axis_index("core")
        base = wid * per_w
        pltpu.sync_copy(i0_hbm.at[pl.ds(base, per_w)], i0_v)
        pltpu.sync_copy(i1_hbm.at[pl.ds(base, per_w)], i1_v)
        for c in range(n_ch):
            sl = pl.ds(c * _CH, _CH)
            cp0 = pltpu.async_copy(x_hbm.at[i0_v.at[sl]], row0_v, sem0)
            cp1 = pltpu.async_copy(x_hbm.at[i1_v.at[sl]], row1_v, sem1)
            cp0.wait()
            cp1.wait()

            @pl.loop(0, _CH)
            def _(r):
                for q in range(d // 16):
                    qs = pl.ds(q * 16, 16)
                    row0_v[r, qs] += row1_v[r, qs]

            pltpu.sync_copy(row0_v, o_hbm.at[pl.ds(base + c * _CH, _CH)])

    return k(opad, pos0, pos1)


def _gmm_body(blk_e_ref, xs_ref, w1_ref, b1_ref, w2_ref, b2_ref, pp_ref,
              o_ref):
    del blk_e_ref
    h = lax.dot_general(xs_ref[...], w1_ref[0],
                        (((1,), (1,)), ((), ())),
                        preferred_element_type=jnp.float32)
    h = jnp.maximum(h + b1_ref[0], 0.0)
    o = lax.dot_general(h, w2_ref[0],
                        (((1,), (1,)), ((), ())),
                        preferred_element_type=jnp.float32)
    o_ref[...] = (o + b2_ref[0]) * pp_ref[...]


def _gmm(blk_e, xs_pad, W1, b1, W2, b2, ppad):
    nblk = blk_e.shape[0]
    np_, d = xs_pad.shape
    h = W1.shape[1]
    grid_spec = pltpu.PrefetchScalarGridSpec(
        num_scalar_prefetch=1,
        grid=(nblk,),
        in_specs=[
            pl.BlockSpec((_T, d), lambda g, be: (g, 0)),
            pl.BlockSpec((1, h, d), lambda g, be: (be[g], 0, 0)),
            pl.BlockSpec((1, 1, h), lambda g, be: (be[g], 0, 0)),
            pl.BlockSpec((1, d, h), lambda g, be: (be[g], 0, 0)),
            pl.BlockSpec((1, 1, d), lambda g, be: (be[g], 0, 0)),
            pl.BlockSpec((_T, 1), lambda g, be: (g, 0)),
        ],
        out_specs=pl.BlockSpec((_T, d), lambda g, be: (g, 0)),
    )
    return pl.pallas_call(
        _gmm_body,
        grid_spec=grid_spec,
        out_shape=jax.ShapeDtypeStruct((np_, d), jnp.float32),
        compiler_params=pltpu.CompilerParams(
            dimension_semantics=("arbitrary",)),
    )(blk_e, xs_pad, W1, b1.reshape(b1.shape[0], 1, h),
      W2, b2.reshape(b2.shape[0], 1, d), ppad)


def kernel(x, Wg, bg, W1, b1, W2, b2):
    bx, sx, d = x.shape
    n = bx * sx
    e = Wg.shape[0]
    kk = 2
    x_flat = x.reshape(n, d)

    i1, i2, p1, p2, imp_sum, load_sum = _gate(x_flat, Wg, bg)

    e_flat = jnp.concatenate([i1, i2], axis=1).reshape(-1)
    probs_flat = jnp.concatenate([p1, p2], axis=1).reshape(-1)
    oh = (e_flat[:, None] == jnp.arange(e, dtype=jnp.int32)[None, :]
          ).astype(jnp.int32)
    cum = jnp.cumsum(oh, axis=0)
    rk = jnp.take_along_axis(cum, e_flat[:, None], axis=1)[:, 0] - 1
    counts = cum[-1]
    nblk_e = (counts + _T - 1) // _T
    cum_nblk = jnp.cumsum(nblk_e)
    bstart = jnp.concatenate(
        [jnp.zeros((1,), jnp.int32), cum_nblk[:-1]])
    nblk = n * kk // _T + e
    blk_e = jnp.minimum(
        jnp.searchsorted(cum_nblk, jnp.arange(nblk, dtype=jnp.int32),
                         side="right"),
        e - 1).astype(jnp.int32)
    pp = bstart[e_flat] * _T + rk
    np_ = nblk * _T
    gidx = jnp.zeros((np_,), jnp.int32).at[pp].set(
        (jnp.arange(n * kk, dtype=jnp.int32) // kk),
        unique_indices=True)
    ppad = jnp.zeros((np_, 1), jnp.float32).at[pp, 0].set(
        probs_flat, unique_indices=True)
    pos = pp.reshape(n, kk)

    xs_pad = _sc_gather(x_flat, gidx)
    opad = _gmm(blk_e, xs_pad, W1, b1, W2, b2, ppad)
    y = _sc_combine(opad, pos[:, 0], pos[:, 1])

    outputs = y.reshape(bx, sx, d)
    nf = jnp.float32(n)
    aux_loss = jnp.sum((imp_sum[0] / nf) * (load_sum[0] / nf)) * e
    return outputs, aux_loss
